# baseline (speedup 1.0000x reference)
import jax
import jax.numpy as jnp
from jax import lax
from jax.experimental import pallas as pl
from jax.experimental.pallas import tpu as pltpu
from jax.experimental.pallas import tpu_sc as plsc

B, T = 1024, 200
VOCAB, D = 100000, 64
EPS = 1e-5

NC, NS, L = 2, 16, 16
NW = NC * NS
TT = T // 8
TB = B // 128
NBLK = TT * TB


def _iota16():
    return lax.iota(jnp.int32, L)


def _fast_rsqrt(x):
    i = plsc.bitcast(x, jnp.int32)
    i = jnp.int32(0x5F3759DF) - lax.shift_right_logical(i, 1)
    y = plsc.bitcast(i, jnp.float32)
    for _ in range(3):
        y = y * (1.5 - 0.5 * x * y * y)
    return y


def _sc_body(cat1_h, cat2_h, num_h, len_h, gb_h, embc_h, out_h,
             ctile1_v, ctile2_v, rows_v, buf3_v, ntile_v, bn_v, len_v,
             gb_v, rd_v, pub_v, shared_v, semg):
    cid = lax.axis_index("c")
    sid = lax.axis_index("s")
    wid = sid * NC + cid
    iota = _iota16()

    pltpu.sync_copy(len_h, len_v)
    pltpu.sync_copy(gb_h, gb_v)

    def cnt_body(i, acc):
        lv = len_v[pl.ds(i * L, L)]
        return acc + jnp.clip(lv, 0, T).astype(jnp.float32)
    cnt_acc = lax.fori_loop(0, B // L, cnt_body, jnp.zeros((L,), jnp.float32))
    cnt_s = jnp.sum(cnt_acc)

    nstat = jnp.where(sid < NBLK % NS, NBLK // NS + 1, NBLK // NS)

    def stat_body(k, carry):
        a_s, a_q = carry
        blk = sid + k * NS
        tt = blk // TB
        bb = blk % TB
        pltpu.sync_copy(num_h.at[pl.ds(tt * 8, 8), pl.ds(bb * 128, 128)],
                        ntile_v)
        for trow in range(8):
            t = tt * 8 + trow
            for g in range(8):
                x = ntile_v[trow, pl.ds(g * 16, 16)]
                lg = len_v[pl.ds(bb * 128 + g * 16, 16)]
                m = lg > t
                a_s = a_s + jnp.where(m, x, 0.0)
                a_q = a_q + jnp.where(m, x * x, 0.0)
        return a_s, a_q

    acc_s, acc_q = lax.fori_loop(
        0, nstat, stat_body,
        (jnp.zeros((L,), jnp.float32), jnp.zeros((L,), jnp.float32)))

    s_part = jnp.sum(acc_s)
    q_part = jnp.sum(acc_q)
    pvec = jnp.where(iota == 0, s_part, jnp.where(iota == 1, q_part, 0.0))
    pub_v[...] = pvec
    pltpu.sync_copy(pub_v, shared_v.at[pl.ds(sid * L, L)])
    plsc.subcore_barrier()
    pltpu.sync_copy(shared_v, rd_v)
    tot = jnp.zeros((L,), jnp.float32)
    for i in range(NS):
        tot = tot + rd_v[pl.ds(i * L, L)]

    def _lane(v, k):
        return jnp.full((L,), jnp.sum(jnp.where(iota == k, v, 0.0)))

    cnt_sp = jnp.maximum(jnp.full((L,), cnt_s), 1.0)
    mean = _lane(tot, 0) / cnt_sp
    var = _lane(tot, 1) / cnt_sp - mean * mean
    inv = _fast_rsqrt(var + EPS)
    gbvec = gb_v[...]
    scale = _lane(gbvec, 0) * inv
    shift = _lane(gbvec, 1) - mean * scale

    nblk = jnp.where(wid < NBLK % NW, NBLK // NW + 1, NBLK // NW)
    cvecs = [iota + c0 for c0 in (0, 16, 32, 48)]

    def blk_body(k, _):
        blk = wid + k * NW
        tt = blk // TB
        bb = blk % TB
        tds = pl.ds(tt * 8, 8)
        bds = pl.ds(bb * 128, 128)
        pltpu.sync_copy(cat1_h.at[tds, bds], ctile1_v)
        pltpu.sync_copy(cat2_h.at[tds, bds], ctile2_v)
        pltpu.sync_copy(num_h.at[tds, bds], ntile_v)

        for trow in range(8):
            t = tt * 8 + trow
            for g in range(8):
                x = ntile_v[trow, pl.ds(g * 16, 16)]
                lg = len_v[pl.ds(bb * 128 + g * 16, 16)]
                m = lg > t
                bn_v[trow, pl.ds(g * 16, 16)] = jnp.where(
                    m, x * scale + shift, x)
        pltpu.sync_copy(bn_v, out_h.at[2 * D, tds, bds])

        for tb, ctile in ((0, ctile1_v), (1, ctile2_v)):
            fb = tb * D

            def row_body(r, _):
                pltpu.async_copy(embc_h.at[ctile.at[r]], rows_v, semg).wait()

                def i_body(i, _):
                    r_sp = jnp.full((L,), r, jnp.int32)
                    i_sp = jnp.full((L,), i, jnp.int32)
                    for c0 in range(4):
                        x = rows_v[i, pl.ds(fb + c0 * 16, 16)]
                        plsc.store_scatter(
                            buf3_v, [cvecs[c0], r_sp, i_sp], x)
                    return ()
                lax.fori_loop(0, 128, i_body, ())
                return ()
            lax.fori_loop(0, 8, row_body, ())
            pltpu.sync_copy(buf3_v, out_h.at[pl.ds(tb * D, D), tds, bds])
        return ()

    lax.fori_loop(0, nblk, blk_body, ())


@jax.jit
def _sc_feature_processor(catT1, catT2, numT, lens, gb, embc):
    mesh = plsc.VectorSubcoreMesh(core_axis_name="c", subcore_axis_name="s")
    kern = pl.kernel(
        _sc_body,
        out_type=jax.ShapeDtypeStruct((2 * D + 1, T, B), jnp.float32),
        mesh=mesh,
        compiler_params=pltpu.CompilerParams(use_tc_tiling_on_sc=True,
                                             needs_layout_passes=False),
        scratch_types=[
            pltpu.VMEM((8, 128), jnp.int32),
            pltpu.VMEM((8, 128), jnp.int32),
            pltpu.VMEM((128, 128), jnp.float32),
            pltpu.VMEM((D, 8, 128), jnp.float32),
            pltpu.VMEM((8, 128), jnp.float32),
            pltpu.VMEM((8, 128), jnp.float32),
            pltpu.VMEM((B,), jnp.int32),
            pltpu.VMEM((L,), jnp.float32),
            pltpu.VMEM((NS * L,), jnp.float32),
            pltpu.VMEM((L,), jnp.float32),
            pltpu.VMEM_SHARED((NS * L,), jnp.float32),
            pltpu.SemaphoreType.DMA,
        ],
    )
    return kern(catT1, catT2, numT, lens, gb, embc)


def kernel(event_time, seq_lens, cat1, cat2, num1, emb1, emb2, gamma, beta):
    catT1 = cat1.astype(jnp.int32).T
    catT2 = cat2.astype(jnp.int32).T
    numT = num1.astype(jnp.float32).T
    lens = seq_lens.astype(jnp.int32)
    gb = jnp.concatenate([gamma.astype(jnp.float32),
                          beta.astype(jnp.float32),
                          jnp.zeros((14,), jnp.float32)])
    embc = jnp.concatenate([emb1, emb2], axis=1)
    out3 = _sc_feature_processor(catT1, catT2, numT, lens, gb, embc)
    return out3.transpose(2, 1, 0), event_time.astype(jnp.float32)


# speedup vs baseline: 1.1729x; 1.1729x over previous
import jax
import jax.numpy as jnp
from jax import lax
from jax.experimental import pallas as pl
from jax.experimental.pallas import tpu as pltpu
from jax.experimental.pallas import tpu_sc as plsc

B, T = 1024, 200
VOCAB, D = 100000, 64
EPS = 1e-5

NC, NS, L = 2, 16, 16
NW = NC * NS
TT = T // 8
TB = B // 128
NBLK = TT * TB


def _iota16():
    return lax.iota(jnp.int32, L)


def _fast_rsqrt(x):
    i = plsc.bitcast(x, jnp.int32)
    i = jnp.int32(0x5F3759DF) - lax.shift_right_logical(i, 1)
    y = plsc.bitcast(i, jnp.float32)
    for _ in range(3):
        y = y * (1.5 - 0.5 * x * y * y)
    return y


def _sc_body(cat1_h, cat2_h, num_h, len_h, gb_h, embc_h, out_h,
             ctile1_v, ctile2_v, rows_v, buf3_v, ntile_v, bn_v, len_v,
             gb_v, rd_v, pub_v, shared_v, semg, semw):
    cid = lax.axis_index("c")
    sid = lax.axis_index("s")
    wid = sid * NC + cid
    iota = _iota16()

    pltpu.sync_copy(len_h, len_v)
    pltpu.sync_copy(gb_h, gb_v)

    def cnt_body(i, acc):
        lv = len_v[pl.ds(i * L, L)]
        return acc + jnp.clip(lv, 0, T).astype(jnp.float32)
    cnt_acc = lax.fori_loop(0, B // L, cnt_body, jnp.zeros((L,), jnp.float32))
    cnt_s = jnp.sum(cnt_acc)

    nstat = jnp.where(sid < NBLK % NS, NBLK // NS + 1, NBLK // NS)

    def stat_body(k, carry):
        a_s, a_q = carry
        blk = sid + k * NS
        tt = blk // TB
        bb = blk % TB
        pltpu.sync_copy(num_h.at[pl.ds(tt * 8, 8), pl.ds(bb * 128, 128)],
                        ntile_v)
        for trow in range(8):
            t = tt * 8 + trow
            for g in range(8):
                x = ntile_v[trow, pl.ds(g * 16, 16)]
                lg = len_v[pl.ds(bb * 128 + g * 16, 16)]
                m = lg > t
                a_s = a_s + jnp.where(m, x, 0.0)
                a_q = a_q + jnp.where(m, x * x, 0.0)
        return a_s, a_q

    acc_s, acc_q = lax.fori_loop(
        0, nstat, stat_body,
        (jnp.zeros((L,), jnp.float32), jnp.zeros((L,), jnp.float32)))

    s_part = jnp.sum(acc_s)
    q_part = jnp.sum(acc_q)
    pvec = jnp.where(iota == 0, s_part, jnp.where(iota == 1, q_part, 0.0))
    pub_v[...] = pvec
    pltpu.sync_copy(pub_v, shared_v.at[pl.ds(sid * L, L)])
    plsc.subcore_barrier()
    pltpu.sync_copy(shared_v, rd_v)
    tot = jnp.zeros((L,), jnp.float32)
    for i in range(NS):
        tot = tot + rd_v[pl.ds(i * L, L)]

    def _lane(v, k):
        return jnp.full((L,), jnp.sum(jnp.where(iota == k, v, 0.0)))

    cnt_sp = jnp.maximum(jnp.full((L,), cnt_s), 1.0)
    mean = _lane(tot, 0) / cnt_sp
    var = _lane(tot, 1) / cnt_sp - mean * mean
    inv = _fast_rsqrt(var + EPS)
    gbvec = gb_v[...]
    scale = _lane(gbvec, 0) * inv
    shift = _lane(gbvec, 1) - mean * scale

    nblk = jnp.where(wid < NBLK % NW, NBLK // NW + 1, NBLK // NW)
    cvecs = [iota + c0 for c0 in (0, 16, 32, 48)]

    def blk_body(k, _):
        blk = wid + k * NW
        tt = blk // TB
        bb = blk % TB
        tds = pl.ds(tt * 8, 8)
        bds = pl.ds(bb * 128, 128)
        pltpu.sync_copy(cat1_h.at[tds, bds], ctile1_v)
        pltpu.sync_copy(cat2_h.at[tds, bds], ctile2_v)
        pltpu.sync_copy(num_h.at[tds, bds], ntile_v)

        for trow in range(8):
            t = tt * 8 + trow
            for g in range(8):
                x = ntile_v[trow, pl.ds(g * 16, 16)]
                lg = len_v[pl.ds(bb * 128 + g * 16, 16)]
                m = lg > t
                bn_v[trow, pl.ds(g * 16, 16)] = jnp.where(
                    m, x * scale + shift, x)
        pltpu.sync_copy(bn_v, out_h.at[2 * D, tds, bds])

        z = blk * 0
        prev_out = []
        for tb, ctile in ((0, ctile1_v), (1, ctile2_v)):
            fb = tb * D
            descs = [None, None]
            descs[0] = pltpu.async_copy(embc_h.at[ctile.at[0]],
                                        rows_v.at[0], semg)
            for r in range(8):
                descs[r % 2].wait()
                if r < 7:
                    descs[(r + 1) % 2] = pltpu.async_copy(
                        embc_h.at[ctile.at[r + 1]], rows_v.at[(r + 1) % 2],
                        semg)
                if prev_out:
                    prev_out.pop().wait()
                rbuf = rows_v.at[r % 2]
                r_sp = jnp.full((L,), z + r, jnp.int32)

                def i_body(ii, _):
                    base = ii * 4
                    for u in range(4):
                        i_sp = jnp.full((L,), base + u, jnp.int32)
                        for c0 in range(4):
                            x = rbuf[base + u, pl.ds(fb + c0 * 16, 16)]
                            plsc.store_scatter(
                                buf3_v, [cvecs[c0], r_sp, i_sp], x)
                    return ()
                lax.fori_loop(0, 32, i_body, ())
            prev_out.append(pltpu.async_copy(
                buf3_v, out_h.at[pl.ds(tb * D, D), tds, bds], semw))
        prev_out.pop().wait()
        return ()

    lax.fori_loop(0, nblk, blk_body, ())


@jax.jit
def _sc_feature_processor(catT1, catT2, numT, lens, gb, embc):
    mesh = plsc.VectorSubcoreMesh(core_axis_name="c", subcore_axis_name="s")
    kern = pl.kernel(
        _sc_body,
        out_type=jax.ShapeDtypeStruct((2 * D + 1, T, B), jnp.float32),
        mesh=mesh,
        compiler_params=pltpu.CompilerParams(use_tc_tiling_on_sc=True,
                                             needs_layout_passes=False),
        scratch_types=[
            pltpu.VMEM((8, 128), jnp.int32),
            pltpu.VMEM((8, 128), jnp.int32),
            pltpu.VMEM((2, 128, 128), jnp.float32),
            pltpu.VMEM((D, 8, 128), jnp.float32),
            pltpu.VMEM((8, 128), jnp.float32),
            pltpu.VMEM((8, 128), jnp.float32),
            pltpu.VMEM((B,), jnp.int32),
            pltpu.VMEM((L,), jnp.float32),
            pltpu.VMEM((NS * L,), jnp.float32),
            pltpu.VMEM((L,), jnp.float32),
            pltpu.VMEM_SHARED((NS * L,), jnp.float32),
            pltpu.SemaphoreType.DMA,
            pltpu.SemaphoreType.DMA,
        ],
    )
    return kern(catT1, catT2, numT, lens, gb, embc)


def kernel(event_time, seq_lens, cat1, cat2, num1, emb1, emb2, gamma, beta):
    catT1 = cat1.astype(jnp.int32).T
    catT2 = cat2.astype(jnp.int32).T
    numT = num1.astype(jnp.float32).T
    lens = seq_lens.astype(jnp.int32)
    gb = jnp.concatenate([gamma.astype(jnp.float32),
                          beta.astype(jnp.float32),
                          jnp.zeros((14,), jnp.float32)])
    embc = jnp.concatenate([emb1, emb2], axis=1)
    out3 = _sc_feature_processor(catT1, catT2, numT, lens, gb, embc)
    return out3.transpose(2, 1, 0), event_time.astype(jnp.float32)


# speedup vs baseline: 1.3874x; 1.1829x over previous
import jax
import jax.numpy as jnp
from jax import lax
from jax.experimental import pallas as pl
from jax.experimental.pallas import tpu as pltpu
from jax.experimental.pallas import tpu_sc as plsc

B, T = 1024, 200
VOCAB, D = 100000, 64
N = B * T
EPS = 1e-5

NC, NS, L = 2, 16, 16
NW = NC * NS
ROWS_W = N // NW
CHUNK = 400
NCHUNK = ROWS_W // CHUNK
G = 100
NG = CHUNK // G
BROW_W = B // NW
STAT_ROWS = B // NS
JCOL = (T + L - 1) // L


def _iota16():
    return lax.iota(jnp.int32, L)


def _splat_i32(x):
    return jnp.full((L,), x, dtype=jnp.int32)


def _fast_rsqrt(x):
    i = plsc.bitcast(x, jnp.int32)
    i = jnp.int32(0x5F3759DF) - lax.shift_right_logical(i, 1)
    y = plsc.bitcast(i, jnp.float32)
    for _ in range(3):
        y = y * (1.5 - 0.5 * x * y * y)
    return y


def _sc_body(cat1_h, cat2_h, num_h, len_h, gb_h, emb1_h, emb2_h, out_h,
             idx1_v, idx2_v, rows1_v, rows2_v, nbuf_v, numc_v, len_v,
             gb_v, stat_v, pub_v, shared_v, semg, semw0, semw1):
    cid = lax.axis_index("c")
    sid = lax.axis_index("s")
    wid = sid * NC + cid
    iota = _iota16()

    pltpu.sync_copy(len_h, len_v)
    pltpu.sync_copy(gb_h, gb_v)

    def cnt_body(i, acc):
        lv = len_v[pl.ds(i * L, L)]
        lc = jnp.clip(lv, 0, T).astype(jnp.float32)
        return acc + lc
    cnt_acc = lax.fori_loop(0, B // L, cnt_body, jnp.zeros((L,), jnp.float32))
    cnt_s = jnp.sum(cnt_acc)

    r0 = sid * STAT_ROWS
    acc_s = jnp.zeros((L,), jnp.float32)
    acc_q = jnp.zeros((L,), jnp.float32)
    SUB = 16
    for sub in range(STAT_ROWS // SUB):
        pltpu.sync_copy(num_h.at[pl.ds((r0 + sub * SUB) * T, SUB * T)],
                        stat_v.at[pl.ds(0, SUB * T)])

        def row_body(r, carry):
            a_s, a_q = carry
            len_sp = plsc.load_gather(len_v, [_splat_i32(r0 + sub * SUB + r)])
            for j in range(JCOL):
                x = stat_v[pl.ds(r * T + j * L, L)]
                col = iota + (j * L)
                m = (col < len_sp) & (col < T)
                a_s = a_s + jnp.where(m, x, 0.0)
                a_q = a_q + jnp.where(m, x * x, 0.0)
            return a_s, a_q
        acc_s, acc_q = lax.fori_loop(0, SUB, row_body, (acc_s, acc_q))

    s_part = jnp.sum(acc_s)
    q_part = jnp.sum(acc_q)
    pvec = jnp.where(iota == 0, s_part, jnp.where(iota == 1, q_part, 0.0))
    pub_v[...] = pvec
    pltpu.sync_copy(pub_v, shared_v.at[pl.ds(sid * L, L)])
    plsc.subcore_barrier()
    pltpu.sync_copy(shared_v, stat_v.at[pl.ds(0, NS * L)])
    tot = jnp.zeros((L,), jnp.float32)
    for i in range(NS):
        tot = tot + stat_v[pl.ds(i * L, L)]
    def _lane(v, k):
        return jnp.full((L,), jnp.sum(jnp.where(iota == k, v, 0.0)))

    sum_sp = _lane(tot, 0)
    q_sp = _lane(tot, 1)
    cnt_sp = jnp.maximum(jnp.full((L,), cnt_s), 1.0)
    mean = sum_sp / cnt_sp
    var = q_sp / cnt_sp - mean * mean
    inv = _fast_rsqrt(var + EPS)
    gbvec = gb_v[...]
    gamma_sp = _lane(gbvec, 0)
    beta_sp = _lane(gbvec, 1)
    scale = gamma_sp * inv
    shift = beta_sp - mean * scale

    semws = (semw0, semw1)
    wpend = [[], []]
    for t in range(NCHUNK):
        p = t % 2
        for d in wpend[p]:
            d.wait()
        wpend[p] = []
        rowbase = wid * ROWS_W + t * CHUNK
        rows1 = rows1_v.at[p]
        rows2 = rows2_v.at[p]
        nbuf = nbuf_v.at[p]
        pltpu.sync_copy(cat1_h.at[pl.ds(wid * (ROWS_W // G) + t * NG, NG)],
                        idx1_v)
        pltpu.sync_copy(cat2_h.at[pl.ds(wid * (ROWS_W // G) + t * NG, NG)],
                        idx2_v)
        copies = []
        for k in range(NG):
            copies.append(pltpu.async_copy(
                emb1_h.at[idx1_v.at[k]], rows1.at[pl.ds(k * G, G)], semg))
        for k in range(NG):
            copies.append(pltpu.async_copy(
                emb2_h.at[idx2_v.at[k]], rows2.at[pl.ds(k * G, G)], semg))

        pltpu.sync_copy(num_h.at[pl.ds(rowbase, CHUNK)],
                        numc_v.at[pl.ds(0, CHUNK)])
        for b in range(CHUNK // T):
            brow = wid * BROW_W + t * (CHUNK // T) + b
            len_sp = plsc.load_gather(len_v, [_splat_i32(brow)])
            for j in range(JCOL):
                x = numc_v[pl.ds(b * T + j * L, L)]
                col = iota + (j * L)
                m = col < len_sp
                val = jnp.where(m, x * scale + shift, x)
                ridx = iota + (b * T + j * L)
                if (j + 1) * L <= T:
                    plsc.store_scatter(nbuf, [ridx, _splat_i32(0)], val)
                else:
                    plsc.store_scatter(nbuf, [ridx, _splat_i32(0)], val,
                                       mask=col < T)

        for c in copies:
            c.wait()
        wpend[p] = [
            pltpu.async_copy(
                rows1, out_h.at[pl.ds(rowbase, CHUNK), pl.ds(0, D)], semws[p]),
            pltpu.async_copy(
                rows2, out_h.at[pl.ds(rowbase, CHUNK), pl.ds(D, D)], semws[p]),
            pltpu.async_copy(
                nbuf, out_h.at[pl.ds(rowbase, CHUNK), pl.ds(2 * D, 1)],
                semws[p]),
        ]
    for p in range(2):
        for d in wpend[p]:
            d.wait()


@jax.jit
def _sc_feature_processor(cat1f, cat2f, num1f, lens, gb, emb1, emb2):
    mesh = plsc.VectorSubcoreMesh(core_axis_name="c", subcore_axis_name="s")
    kern = pl.kernel(
        _sc_body,
        out_type=jax.ShapeDtypeStruct((N, 2 * D + 1), jnp.float32),
        mesh=mesh,
        compiler_params=pltpu.CompilerParams(use_tc_tiling_on_sc=False,
                                             needs_layout_passes=False),
        scratch_types=[
            pltpu.VMEM((NG, G), jnp.int32),
            pltpu.VMEM((NG, G), jnp.int32),
            pltpu.VMEM((2, CHUNK, D), jnp.float32),
            pltpu.VMEM((2, CHUNK, D), jnp.float32),
            pltpu.VMEM((2, CHUNK, 1), jnp.float32),
            pltpu.VMEM((CHUNK + L, ), jnp.float32),
            pltpu.VMEM((B,), jnp.int32),
            pltpu.VMEM((L,), jnp.float32),
            pltpu.VMEM((16 * T + L,), jnp.float32),
            pltpu.VMEM((L,), jnp.float32),
            pltpu.VMEM_SHARED((NS * L,), jnp.float32),
            pltpu.SemaphoreType.DMA,
            pltpu.SemaphoreType.DMA,
            pltpu.SemaphoreType.DMA,
        ],
    )
    return kern(cat1f, cat2f, num1f, lens, gb, emb1, emb2)


def kernel(event_time, seq_lens, cat1, cat2, num1, emb1, emb2, gamma, beta):
    cat1f = cat1.reshape(N // G, G).astype(jnp.int32)
    cat2f = cat2.reshape(N // G, G).astype(jnp.int32)
    num1f = num1.astype(jnp.float32).reshape(N)
    lens = seq_lens.astype(jnp.int32)
    gb = jnp.concatenate([gamma.astype(jnp.float32),
                          beta.astype(jnp.float32),
                          jnp.zeros((14,), jnp.float32)])
    out = _sc_feature_processor(cat1f, cat2f, num1f, lens, gb, emb1, emb2)
    return out.reshape(B, T, 2 * D + 1), event_time.astype(jnp.float32)
